# baseline (speedup 1.0000x reference)
import functools

import jax
import jax.numpy as jnp
from jax import lax
from jax.experimental import pallas as pl
from jax.experimental.pallas import tpu as pltpu
from jax.experimental.pallas import tpu_sc as plsc

DIM = 2048
B_TOT = 4 * 4096
NC, NS = 2, 16
NW = NC * NS
B_PER_W = B_TOT // NW
CHUNK = 16
N_CHUNKS = B_PER_W // CHUNK

_mesh = plsc.VectorSubcoreMesh(core_axis_name="c", subcore_axis_name="s")


@functools.partial(
    pl.kernel,
    out_type=jax.ShapeDtypeStruct((B_TOT, DIM), jnp.float32),
    mesh=_mesh,
    scratch_types=[
        pltpu.VMEM((N_CHUNKS, CHUNK), jnp.int32),
        pltpu.VMEM((CHUNK, DIM), jnp.float32),
        pltpu.VMEM((CHUNK, DIM), jnp.float32),
        pltpu.VMEM((CHUNK, DIM), jnp.float32),
        pltpu.SemaphoreType.DMA,
        pltpu.SemaphoreType.DMA,
        pltpu.SemaphoreType.DMA,
    ],
)
def _emb_lookup(idx_hbm, table_hbm, out_hbm, idx_v, rows0, rows1, rows2,
                gsem0, gsem1, gsem2):
    wid = lax.axis_index("s") * NC + lax.axis_index("c")
    base = wid * B_PER_W
    bufs = ((rows0, gsem0), (rows1, gsem1), (rows2, gsem2))

    pltpu.sync_copy(idx_hbm.at[wid], idx_v)

    def start_gather(c, buf, sem):
        pltpu.async_copy(table_hbm.at[idx_v.at[c]], buf, sem)

    def wait_gather(buf, sem):
        pltpu.make_async_copy(table_hbm.at[idx_v.at[0]], buf, sem).wait()

    def store(buf, c):
        pltpu.sync_copy(buf, out_hbm.at[pl.ds(base + c * CHUNK, CHUNK)])

    for b in range(3):
        start_gather(b, *bufs[b])

    def body(g, carry):
        for b in range(3):
            c = 3 * g + b
            buf, sem = bufs[b]
            wait_gather(buf, sem)
            store(buf, c)

            @pl.when(c + 3 < N_CHUNKS)
            def _():
                start_gather(c + 3, buf, sem)

        return carry

    lax.fori_loop(0, N_CHUNKS // 3, body, 0)
    for b in range(N_CHUNKS % 3):
        c = (N_CHUNKS // 3) * 3 + b
        buf, sem = bufs[b]
        wait_gather(buf, sem)
        store(buf, c)


def kernel(x, W):
    idx = x.reshape(NW, N_CHUNKS, CHUNK).astype(jnp.int32)
    out = _emb_lookup(idx, W)
    return out.reshape(x.shape[0], x.shape[1], DIM)


# speedup vs baseline: 1.0023x; 1.0023x over previous
import functools

import jax
import jax.numpy as jnp
from jax import lax
from jax.experimental import pallas as pl
from jax.experimental.pallas import tpu as pltpu
from jax.experimental.pallas import tpu_sc as plsc

DIM = 2048
B_TOT = 4 * 4096
NC, NS = 2, 16
NW = NC * NS
B_PER_W = B_TOT // NW
CHUNK = 16
N_CHUNKS = B_PER_W // CHUNK

_mesh = plsc.VectorSubcoreMesh(core_axis_name="c", subcore_axis_name="s")


@functools.partial(
    pl.kernel,
    out_type=jax.ShapeDtypeStruct((B_TOT, DIM), jnp.float32),
    mesh=_mesh,
    scratch_types=[
        pltpu.VMEM((B_PER_W,), jnp.int32),
        pltpu.VMEM((CHUNK, DIM), jnp.float32),
        pltpu.VMEM((CHUNK, DIM), jnp.float32),
        pltpu.SemaphoreType.DMA,
        pltpu.SemaphoreType.DMA,
    ],
)
def _emb_lookup(idx_hbm, table_hbm, out_hbm, idx_v, rows0, rows1, gsem0, gsem1):
    wid = lax.axis_index("s") * NC + lax.axis_index("c")
    base = wid * B_PER_W

    pltpu.sync_copy(idx_hbm.at[wid // 8, pl.ds((wid % 8) * B_PER_W, B_PER_W)],
                    idx_v)

    def start_gather(c, buf, sem):
        pltpu.async_copy(table_hbm.at[idx_v.at[pl.ds(c * CHUNK, CHUNK)]],
                         buf, sem)

    def wait_gather(buf, sem):
        pltpu.make_async_copy(table_hbm.at[idx_v.at[pl.ds(0, CHUNK)]],
                              buf, sem).wait()

    def store(buf, c):
        pltpu.sync_copy(buf, out_hbm.at[pl.ds(base + c * CHUNK, CHUNK)])

    start_gather(0, rows0, gsem0)

    def body(i, carry):
        c = 2 * i
        start_gather(c + 1, rows1, gsem1)
        wait_gather(rows0, gsem0)
        store(rows0, c)

        @pl.when(c + 2 < N_CHUNKS)
        def _():
            start_gather(c + 2, rows0, gsem0)

        wait_gather(rows1, gsem1)
        store(rows1, c + 1)
        return carry

    lax.fori_loop(0, N_CHUNKS // 2, body, 0)


def kernel(x, W):
    out = _emb_lookup(x.astype(jnp.int32), W)
    return out.reshape(x.shape[0], x.shape[1], DIM)
